# baseline (speedup 1.0000x reference)
import functools

import jax
import jax.numpy as jnp
from jax import lax
from jax.experimental import pallas as pl
from jax.experimental.pallas import tpu as pltpu
from jax.experimental.pallas import tpu_sc as plsc

_VOCAB = 49408
_DIM = 1280
_NIDX = 16
_NW = 32
_ROWS_PER_W = _VOCAB // _NW


def _sc_body(emb_hbm, tok_hbm, idx_hbm, out_hbm, idx_v, rows_v, tok_v, sem):
    c = lax.axis_index("c")
    s = lax.axis_index("s")
    wid = s * 2 + c
    base = wid * _ROWS_PER_W

    pltpu.sync_copy(
        emb_hbm.at[pl.ds(base, _ROWS_PER_W)],
        out_hbm.at[pl.ds(base, _ROWS_PER_W)],
    )

    pltpu.sync_copy(idx_hbm, idx_v)
    pltpu.async_copy(emb_hbm.at[idx_v], rows_v, sem).wait()
    pltpu.sync_copy(tok_hbm, tok_v)

    for i in range(_NIDX):
        def _add(d, carry, i=i):
            rows_v[i, pl.ds(d * 16, 16)] = (
                rows_v[i, pl.ds(d * 16, 16)] + tok_v[i, pl.ds(d * 16, 16)]
            )
            return carry
        lax.fori_loop(0, _DIM // 16, _add, 0)

    iv = idx_v[...]
    for i in range(_NIDX):
        idx_i = iv[i]

        @pl.when(jnp.logical_and(idx_i >= base, idx_i < base + _ROWS_PER_W))
        def _(i=i, idx_i=idx_i):
            pltpu.sync_copy(
                rows_v.at[pl.ds(i, 1)], out_hbm.at[pl.ds(idx_i, 1)]
            )


def kernel(embeddings, token_embeddings, indices):
    mesh = plsc.VectorSubcoreMesh(core_axis_name="c", subcore_axis_name="s")
    run = functools.partial(
        pl.kernel,
        out_type=jax.ShapeDtypeStruct((_VOCAB, _DIM), jnp.float32),
        mesh=mesh,
        scratch_types=[
            pltpu.VMEM((_NIDX,), jnp.int32),
            pltpu.VMEM((_NIDX, _DIM), jnp.float32),
            pltpu.VMEM((_NIDX, _DIM), jnp.float32),
            pltpu.SemaphoreType.DMA,
        ],
    )(_sc_body)
    return run(embeddings, token_embeddings, indices)


# speedup vs baseline: 1.0000x; 1.0000x over previous
import functools

import jax
import jax.numpy as jnp
from jax import lax
from jax.experimental import pallas as pl
from jax.experimental.pallas import tpu as pltpu
from jax.experimental.pallas import tpu_sc as plsc

_VOCAB = 49408
_DIM = 1280
_NIDX = 16
_NW = 32
_ROWS_PER_W = _VOCAB // _NW


def _sc_body(emb_hbm, tok_hbm, idx_hbm, out_hbm, idx_v, rows_v, tok_v, sem):
    c = lax.axis_index("c")
    s = lax.axis_index("s")
    wid = s * 2 + c
    base = wid * _ROWS_PER_W

    _SPLITS = [192] * 8 + [8]
    copies = []
    off = 0
    for sub in _SPLITS:
        copies.append(
            pltpu.async_copy(
                emb_hbm.at[pl.ds(base + off, sub)],
                out_hbm.at[pl.ds(base + off, sub)],
                sem,
            )
        )
        off += sub
    for cp in copies:
        cp.wait()

    pltpu.sync_copy(idx_hbm, idx_v)
    pltpu.async_copy(emb_hbm.at[idx_v], rows_v, sem).wait()
    pltpu.sync_copy(tok_hbm, tok_v)

    for i in range(_NIDX):
        def _add(d, carry, i=i):
            rows_v[i, pl.ds(d * 16, 16)] = (
                rows_v[i, pl.ds(d * 16, 16)] + tok_v[i, pl.ds(d * 16, 16)]
            )
            return carry
        lax.fori_loop(0, _DIM // 16, _add, 0)

    iv = idx_v[...]
    for i in range(_NIDX):
        idx_i = iv[i]

        @pl.when(jnp.logical_and(idx_i >= base, idx_i < base + _ROWS_PER_W))
        def _(i=i, idx_i=idx_i):
            pltpu.sync_copy(
                rows_v.at[pl.ds(i, 1)], out_hbm.at[pl.ds(idx_i, 1)]
            )


def kernel(embeddings, token_embeddings, indices):
    mesh = plsc.VectorSubcoreMesh(core_axis_name="c", subcore_axis_name="s")
    run = functools.partial(
        pl.kernel,
        out_type=jax.ShapeDtypeStruct((_VOCAB, _DIM), jnp.float32),
        mesh=mesh,
        scratch_types=[
            pltpu.VMEM((_NIDX,), jnp.int32),
            pltpu.VMEM((_NIDX, _DIM), jnp.float32),
            pltpu.VMEM((_NIDX, _DIM), jnp.float32),
            pltpu.SemaphoreType.DMA,
        ],
    )(_sc_body)
    return run(embeddings, token_embeddings, indices)


# speedup vs baseline: 43.2634x; 43.2633x over previous
import functools

import jax
import jax.numpy as jnp
from jax import lax
from jax.experimental import pallas as pl
from jax.experimental.pallas import tpu as pltpu
from jax.experimental.pallas import tpu_sc as plsc

_VOCAB = 49408
_DIM = 1280
_NIDX = 16
_BLOCK_ROWS = 2560
_NBLOCKS = (_VOCAB + _BLOCK_ROWS - 1) // _BLOCK_ROWS


def _sc_rows_body(emb_hbm, tok_hbm, idx_hbm, new_hbm, idx_v, row_v, tok_v, sem):
    c = lax.axis_index("c")
    s = lax.axis_index("s")
    wid = s * 2 + c

    @pl.when(wid < _NIDX)
    def _():
        pltpu.sync_copy(idx_hbm, idx_v)
        iv = idx_v[...]
        for i in range(_NIDX):
            @pl.when(wid == i)
            def _(i=i):
                idx_i = iv[i]
                pltpu.async_copy(emb_hbm.at[pl.ds(idx_i, 1)], row_v, sem).wait()
                pltpu.sync_copy(tok_hbm.at[pl.ds(i, 1)], tok_v)

                def _add(d, carry):
                    row_v[0, pl.ds(d * 16, 16)] = (
                        row_v[0, pl.ds(d * 16, 16)] + tok_v[0, pl.ds(d * 16, 16)]
                    )
                    return carry

                lax.fori_loop(0, _DIM // 16, _add, 0)
                pltpu.sync_copy(row_v, new_hbm.at[pl.ds(i, 1)])


def _sc_new_rows(embeddings, token_embeddings, indices):
    mesh = plsc.VectorSubcoreMesh(core_axis_name="c", subcore_axis_name="s")
    run = functools.partial(
        pl.kernel,
        out_type=jax.ShapeDtypeStruct((_NIDX, _DIM), jnp.float32),
        mesh=mesh,
        scratch_types=[
            pltpu.VMEM((_NIDX,), jnp.int32),
            pltpu.VMEM((1, _DIM), jnp.float32),
            pltpu.VMEM((1, _DIM), jnp.float32),
            pltpu.SemaphoreType.DMA,
        ],
    )(_sc_rows_body)
    return run(embeddings, token_embeddings, indices)


def _tc_body(idx_ref, in_ref, new_ref, out_ref):
    out_ref[...] = in_ref[...]
    base = pl.program_id(0) * _BLOCK_ROWS
    for i in range(_NIDX):
        idx = idx_ref[i]
        local = idx - base

        @pl.when(jnp.logical_and(idx >= base, idx < base + _BLOCK_ROWS))
        def _():
            out_ref[pl.ds(local, 1), :] = (
                out_ref[pl.ds(local, 1), :]
                + (new_ref[pl.ds(i, 1), :] - in_ref[pl.ds(local, 1), :])
            )


def kernel(embeddings, token_embeddings, indices):
    new_rows = _sc_new_rows(embeddings, token_embeddings, indices)
    return pl.pallas_call(
        _tc_body,
        grid=(_NBLOCKS,),
        in_specs=[
            pl.BlockSpec(memory_space=pltpu.SMEM),
            pl.BlockSpec((_BLOCK_ROWS, _DIM), lambda i: (i, 0)),
            pl.BlockSpec((_NIDX, _DIM), lambda i: (0, 0)),
        ],
        out_specs=pl.BlockSpec((_BLOCK_ROWS, _DIM), lambda i: (i, 0)),
        out_shape=jax.ShapeDtypeStruct((_VOCAB, _DIM), jnp.float32),
    )(indices, embeddings, new_rows)
